# baseline (speedup 1.0000x reference)
import jax
import jax.numpy as jnp
from jax import lax
from jax.experimental import pallas as pl
from jax.experimental.pallas import tpu as pltpu

_BE = 640
_BN = 2048
_NPAD = 10240


def _onehot(idx_col, base, bn):
    cols = base + lax.broadcasted_iota(jnp.int32, (1, bn), 1)
    return (idx_col == cols).astype(jnp.float32)


def _deg_body(src_ref, dst_ref, degs_ref, degd_ref):
    base = pl.program_id(0) * _BN
    ohs = _onehot(src_ref[...], base, _BN)
    ohd = _onehot(dst_ref[...], base, _BN)
    cs = jnp.sum(ohs, axis=0)[:, None]
    cd = jnp.sum(ohd, axis=0)[:, None]

    @pl.when(pl.program_id(1) == 0)
    def _init():
        degs_ref[...] = cs
        degd_ref[...] = cd

    @pl.when(pl.program_id(1) > 0)
    def _acc():
        degs_ref[...] = degs_ref[...] + cs
        degd_ref[...] = degd_ref[...] + cd


def _deg_call(src_col, dst_col):
    e = src_col.shape[0]
    return pl.pallas_call(
        _deg_body,
        grid=(_NPAD // _BN, e // _BE),
        in_specs=[
            pl.BlockSpec((_BE, 1), lambda i, j: (j, 0)),
            pl.BlockSpec((_BE, 1), lambda i, j: (j, 0)),
        ],
        out_specs=[
            pl.BlockSpec((_BN, 1), lambda i, j: (i, 0)),
            pl.BlockSpec((_BN, 1), lambda i, j: (i, 0)),
        ],
        out_shape=[
            jax.ShapeDtypeStruct((_NPAD, 1), jnp.float32),
            jax.ShapeDtypeStruct((_NPAD, 1), jnp.float32),
        ],
        compiler_params=pltpu.CompilerParams(
            dimension_semantics=("parallel", "arbitrary")),
    )(src_col, dst_col)


def _gather_body(src_ref, h_ref, o_ref):
    base = pl.program_id(1) * _BN
    oh = _onehot(src_ref[...], base, _BN)
    c = jnp.dot(oh, h_ref[...], preferred_element_type=jnp.float32)

    @pl.when(pl.program_id(1) == 0)
    def _init():
        o_ref[...] = c

    @pl.when(pl.program_id(1) > 0)
    def _acc():
        o_ref[...] = o_ref[...] + c


def _gather_call(src_col, h_pad):
    e = src_col.shape[0]
    d = h_pad.shape[1]
    return pl.pallas_call(
        _gather_body,
        grid=(e // _BE, _NPAD // _BN),
        in_specs=[
            pl.BlockSpec((_BE, 1), lambda i, j: (i, 0)),
            pl.BlockSpec((_BN, d), lambda i, j: (j, 0)),
        ],
        out_specs=pl.BlockSpec((_BE, d), lambda i, j: (i, 0)),
        out_shape=jax.ShapeDtypeStruct((e, d), jnp.float32),
        compiler_params=pltpu.CompilerParams(
            dimension_semantics=("parallel", "arbitrary")),
    )(src_col, h_pad)


def _scatter_body(dst_ref, hg_ref, o_ref):
    base = pl.program_id(0) * _BN
    oh = _onehot(dst_ref[...], base, _BN)
    c = lax.dot_general(oh, hg_ref[...], (((0,), (0,)), ((), ())),
                        preferred_element_type=jnp.float32)

    @pl.when(pl.program_id(1) == 0)
    def _init():
        o_ref[...] = c

    @pl.when(pl.program_id(1) > 0)
    def _acc():
        o_ref[...] = o_ref[...] + c


def _scatter_call(dst_col, hg):
    e, d = hg.shape
    return pl.pallas_call(
        _scatter_body,
        grid=(_NPAD // _BN, e // _BE),
        in_specs=[
            pl.BlockSpec((_BE, 1), lambda i, j: (j, 0)),
            pl.BlockSpec((_BE, d), lambda i, j: (j, 0)),
        ],
        out_specs=pl.BlockSpec((_BN, d), lambda i, j: (i, 0)),
        out_shape=jax.ShapeDtypeStruct((_NPAD, d), jnp.float32),
        compiler_params=pltpu.CompilerParams(
            dimension_semantics=("parallel", "arbitrary")),
    )(dst_col, hg)


def _prep_body(degs_ref, degd_ref, x_ref, ns_ref, nd_ref, h0s_ref):
    ds_ = degs_ref[...]
    dd_ = degd_ref[...]
    ns = jnp.where(ds_ > 0, 1.0 / jnp.sqrt(jnp.maximum(ds_, 1.0)), 0.0)
    nd_ = jnp.where(dd_ > 0, 1.0 / jnp.sqrt(jnp.maximum(dd_, 1.0)), 0.0)
    ns_ref[...] = ns
    nd_ref[...] = nd_
    h0s_ref[...] = x_ref[...] * ns


def _prep_call(degs, degd, x, bn):
    n, d = x.shape
    return pl.pallas_call(
        _prep_body,
        grid=(n // bn,),
        in_specs=[
            pl.BlockSpec((bn, 1), lambda i: (i, 0)),
            pl.BlockSpec((bn, 1), lambda i: (i, 0)),
            pl.BlockSpec((bn, d), lambda i: (i, 0)),
        ],
        out_specs=[
            pl.BlockSpec((bn, 1), lambda i: (i, 0)),
            pl.BlockSpec((bn, 1), lambda i: (i, 0)),
            pl.BlockSpec((bn, d), lambda i: (i, 0)),
        ],
        out_shape=[
            jax.ShapeDtypeStruct((n, 1), jnp.float32),
            jax.ShapeDtypeStruct((n, 1), jnp.float32),
            jax.ShapeDtypeStruct((n, d), jnp.float32),
        ],
    )(degs, degd, x)


def _layer_body(p_ref, nd_ref, ns_ref, w_ref, b_ref, o_ref):
    agg = p_ref[...] * nd_ref[...]
    t = jnp.dot(agg, w_ref[...], preferred_element_type=jnp.float32)
    t = t + b_ref[...]
    o_ref[...] = jnp.maximum(t, 0.0) * ns_ref[...]


def _layer_call(p, nd_, ns, w, b, bn):
    n, d = p.shape
    return pl.pallas_call(
        _layer_body,
        grid=(n // bn,),
        in_specs=[
            pl.BlockSpec((bn, d), lambda i: (i, 0)),
            pl.BlockSpec((bn, 1), lambda i: (i, 0)),
            pl.BlockSpec((bn, 1), lambda i: (i, 0)),
            pl.BlockSpec((d, d), lambda i: (0, 0)),
            pl.BlockSpec((1, d), lambda i: (0, 0)),
        ],
        out_specs=pl.BlockSpec((bn, d), lambda i: (i, 0)),
        out_shape=jax.ShapeDtypeStruct((n, d), jnp.float32),
    )(p, nd_, ns, w, b)


def _final_body(p_ref, nd_ref, w_ref, b_ref, wfc_ref, bfc_ref, o_ref):
    agg = p_ref[...] * nd_ref[...]
    t = jnp.dot(agg, w_ref[...], preferred_element_type=jnp.float32)
    g = jnp.maximum(t + b_ref[...], 0.0)
    contrib = jnp.sum(g * wfc_ref[...], axis=0, keepdims=True)

    @pl.when(pl.program_id(0) == 0)
    def _init():
        o_ref[...] = bfc_ref[...] + contrib

    @pl.when(pl.program_id(0) > 0)
    def _accum():
        o_ref[...] = o_ref[...] + contrib


def _final_call(p, nd_, w, b, wfc_col, bfc, bn):
    n, d = p.shape
    return pl.pallas_call(
        _final_body,
        grid=(n // bn,),
        in_specs=[
            pl.BlockSpec((bn, d), lambda i: (i, 0)),
            pl.BlockSpec((bn, 1), lambda i: (i, 0)),
            pl.BlockSpec((d, d), lambda i: (0, 0)),
            pl.BlockSpec((1, d), lambda i: (0, 0)),
            pl.BlockSpec((bn, 1), lambda i: (i, 0)),
            pl.BlockSpec((1, 1), lambda i: (0, 0)),
        ],
        out_specs=pl.BlockSpec((1, d), lambda i: (0, 0)),
        out_shape=jax.ShapeDtypeStruct((1, d), jnp.float32),
        compiler_params=pltpu.CompilerParams(
            dimension_semantics=("arbitrary",)),
    )(p, nd_, w, b, wfc_col, bfc)


def kernel(inputs, edge_index, W1, b1, W2, b2, Wfc, bfc):
    x = inputs
    n, d = x.shape
    src_col = edge_index[0].astype(jnp.int32).reshape(-1, 1)
    dst_col = edge_index[1].astype(jnp.int32).reshape(-1, 1)
    bn = 1000
    pad = _NPAD - n

    degs, degd = _deg_call(src_col, dst_col)
    ns, nd_, h0s = _prep_call(degs[:n], degd[:n], x, bn)

    h0p = jnp.concatenate([h0s, jnp.zeros((pad, d), jnp.float32)], axis=0)
    p1 = _scatter_call(dst_col, _gather_call(src_col, h0p))[:n]
    h1s = _layer_call(p1, nd_, ns, W1, b1.reshape(1, d), bn)

    h1p = jnp.concatenate([h1s, jnp.zeros((pad, d), jnp.float32)], axis=0)
    p2 = _scatter_call(dst_col, _gather_call(src_col, h1p))[:n]
    out_row = _final_call(p2, nd_, W2, b2.reshape(1, d),
                          Wfc.reshape(n, 1), bfc.reshape(1, 1), bn)
    return out_row.reshape(d, 1)


# speedup vs baseline: 1.0786x; 1.0786x over previous
import jax
import jax.numpy as jnp
from jax import lax
from jax.experimental import pallas as pl
from jax.experimental.pallas import tpu as pltpu

_BE = 640
_BN = 2048
_NPAD = 10240


def _onehot(idx_col, base, bn, dtype=jnp.float32):
    cols = base + lax.broadcasted_iota(jnp.int32, (1, bn), 1)
    return (idx_col == cols).astype(dtype)


def _deg_body(src_ref, dst_ref, degs_ref, degd_ref):
    base = pl.program_id(0) * _BN
    ohs = _onehot(src_ref[...], base, _BN)
    ohd = _onehot(dst_ref[...], base, _BN)
    cs = jnp.sum(ohs, axis=0)[:, None]
    cd = jnp.sum(ohd, axis=0)[:, None]

    @pl.when(pl.program_id(1) == 0)
    def _init():
        degs_ref[...] = cs
        degd_ref[...] = cd

    @pl.when(pl.program_id(1) > 0)
    def _acc():
        degs_ref[...] = degs_ref[...] + cs
        degd_ref[...] = degd_ref[...] + cd


def _deg_call(src_col, dst_col):
    e = src_col.shape[0]
    return pl.pallas_call(
        _deg_body,
        grid=(_NPAD // _BN, e // _BE),
        in_specs=[
            pl.BlockSpec((_BE, 1), lambda i, j: (j, 0)),
            pl.BlockSpec((_BE, 1), lambda i, j: (j, 0)),
        ],
        out_specs=[
            pl.BlockSpec((_BN, 1), lambda i, j: (i, 0)),
            pl.BlockSpec((_BN, 1), lambda i, j: (i, 0)),
        ],
        out_shape=[
            jax.ShapeDtypeStruct((_NPAD, 1), jnp.float32),
            jax.ShapeDtypeStruct((_NPAD, 1), jnp.float32),
        ],
        compiler_params=pltpu.CompilerParams(
            dimension_semantics=("parallel", "arbitrary")),
    )(src_col, dst_col)


def _gather_body(src_ref, h_ref, o_ref):
    base = pl.program_id(1) * _BN
    oh = _onehot(src_ref[...], base, _BN, jnp.bfloat16)
    c = jnp.dot(oh, h_ref[...], preferred_element_type=jnp.float32)
    c = c.astype(jnp.bfloat16)

    @pl.when(pl.program_id(1) == 0)
    def _init():
        o_ref[...] = c

    @pl.when(pl.program_id(1) > 0)
    def _acc():
        o_ref[...] = o_ref[...] + c


def _gather_call(src_col, h_pad):
    e = src_col.shape[0]
    d = h_pad.shape[1]
    return pl.pallas_call(
        _gather_body,
        grid=(e // _BE, _NPAD // _BN),
        in_specs=[
            pl.BlockSpec((_BE, 1), lambda i, j: (i, 0)),
            pl.BlockSpec((_BN, d), lambda i, j: (j, 0)),
        ],
        out_specs=pl.BlockSpec((_BE, d), lambda i, j: (i, 0)),
        out_shape=jax.ShapeDtypeStruct((e, d), jnp.bfloat16),
        compiler_params=pltpu.CompilerParams(
            dimension_semantics=("parallel", "arbitrary")),
    )(src_col, h_pad)


def _scatter_body(dst_ref, hg_ref, o_ref):
    base = pl.program_id(0) * _BN
    oh = _onehot(dst_ref[...], base, _BN, jnp.bfloat16)
    c = lax.dot_general(oh, hg_ref[...], (((0,), (0,)), ((), ())),
                        preferred_element_type=jnp.float32)

    @pl.when(pl.program_id(1) == 0)
    def _init():
        o_ref[...] = c

    @pl.when(pl.program_id(1) > 0)
    def _acc():
        o_ref[...] = o_ref[...] + c


def _scatter_call(dst_col, hg):
    e, d = hg.shape
    return pl.pallas_call(
        _scatter_body,
        grid=(_NPAD // _BN, e // _BE),
        in_specs=[
            pl.BlockSpec((_BE, 1), lambda i, j: (j, 0)),
            pl.BlockSpec((_BE, d), lambda i, j: (j, 0)),
        ],
        out_specs=pl.BlockSpec((_BN, d), lambda i, j: (i, 0)),
        out_shape=jax.ShapeDtypeStruct((_NPAD, d), jnp.float32),
        compiler_params=pltpu.CompilerParams(
            dimension_semantics=("parallel", "arbitrary")),
    )(dst_col, hg)


def _prep_body(degs_ref, degd_ref, x_ref, ns_ref, nd_ref, h0s_ref):
    ds_ = degs_ref[...]
    dd_ = degd_ref[...]
    ns = jnp.where(ds_ > 0, 1.0 / jnp.sqrt(jnp.maximum(ds_, 1.0)), 0.0)
    nd_ = jnp.where(dd_ > 0, 1.0 / jnp.sqrt(jnp.maximum(dd_, 1.0)), 0.0)
    ns_ref[...] = ns
    nd_ref[...] = nd_
    h0s_ref[...] = x_ref[...] * ns


def _prep_call(degs, degd, x, bn):
    n, d = x.shape
    return pl.pallas_call(
        _prep_body,
        grid=(n // bn,),
        in_specs=[
            pl.BlockSpec((bn, 1), lambda i: (i, 0)),
            pl.BlockSpec((bn, 1), lambda i: (i, 0)),
            pl.BlockSpec((bn, d), lambda i: (i, 0)),
        ],
        out_specs=[
            pl.BlockSpec((bn, 1), lambda i: (i, 0)),
            pl.BlockSpec((bn, 1), lambda i: (i, 0)),
            pl.BlockSpec((bn, d), lambda i: (i, 0)),
        ],
        out_shape=[
            jax.ShapeDtypeStruct((n, 1), jnp.float32),
            jax.ShapeDtypeStruct((n, 1), jnp.float32),
            jax.ShapeDtypeStruct((n, d), jnp.float32),
        ],
    )(degs, degd, x)


def _layer_body(p_ref, nd_ref, ns_ref, w_ref, b_ref, o_ref):
    agg = p_ref[...] * nd_ref[...]
    t = jnp.dot(agg, w_ref[...], preferred_element_type=jnp.float32)
    t = t + b_ref[...]
    o_ref[...] = jnp.maximum(t, 0.0) * ns_ref[...]


def _layer_call(p, nd_, ns, w, b, bn):
    n, d = p.shape
    return pl.pallas_call(
        _layer_body,
        grid=(n // bn,),
        in_specs=[
            pl.BlockSpec((bn, d), lambda i: (i, 0)),
            pl.BlockSpec((bn, 1), lambda i: (i, 0)),
            pl.BlockSpec((bn, 1), lambda i: (i, 0)),
            pl.BlockSpec((d, d), lambda i: (0, 0)),
            pl.BlockSpec((1, d), lambda i: (0, 0)),
        ],
        out_specs=pl.BlockSpec((bn, d), lambda i: (i, 0)),
        out_shape=jax.ShapeDtypeStruct((n, d), jnp.float32),
    )(p, nd_, ns, w, b)


def _final_body(p_ref, nd_ref, w_ref, b_ref, wfc_ref, bfc_ref, o_ref):
    agg = p_ref[...] * nd_ref[...]
    t = jnp.dot(agg, w_ref[...], preferred_element_type=jnp.float32)
    g = jnp.maximum(t + b_ref[...], 0.0)
    contrib = jnp.sum(g * wfc_ref[...], axis=0, keepdims=True)

    @pl.when(pl.program_id(0) == 0)
    def _init():
        o_ref[...] = bfc_ref[...] + contrib

    @pl.when(pl.program_id(0) > 0)
    def _accum():
        o_ref[...] = o_ref[...] + contrib


def _final_call(p, nd_, w, b, wfc_col, bfc, bn):
    n, d = p.shape
    return pl.pallas_call(
        _final_body,
        grid=(n // bn,),
        in_specs=[
            pl.BlockSpec((bn, d), lambda i: (i, 0)),
            pl.BlockSpec((bn, 1), lambda i: (i, 0)),
            pl.BlockSpec((d, d), lambda i: (0, 0)),
            pl.BlockSpec((1, d), lambda i: (0, 0)),
            pl.BlockSpec((bn, 1), lambda i: (i, 0)),
            pl.BlockSpec((1, 1), lambda i: (0, 0)),
        ],
        out_specs=pl.BlockSpec((1, d), lambda i: (0, 0)),
        out_shape=jax.ShapeDtypeStruct((1, d), jnp.float32),
        compiler_params=pltpu.CompilerParams(
            dimension_semantics=("arbitrary",)),
    )(p, nd_, w, b, wfc_col, bfc)


def kernel(inputs, edge_index, W1, b1, W2, b2, Wfc, bfc):
    x = inputs
    n, d = x.shape
    src_col = edge_index[0].astype(jnp.int32).reshape(-1, 1)
    dst_col = edge_index[1].astype(jnp.int32).reshape(-1, 1)
    bn = 1000
    pad = _NPAD - n

    degs, degd = _deg_call(src_col, dst_col)
    ns, nd_, h0s = _prep_call(degs[:n], degd[:n], x, bn)

    h0p = jnp.concatenate([h0s, jnp.zeros((pad, d), jnp.float32)],
                          axis=0).astype(jnp.bfloat16)
    p1 = _scatter_call(dst_col, _gather_call(src_col, h0p))[:n]
    h1s = _layer_call(p1, nd_, ns, W1, b1.reshape(1, d), bn)

    h1p = jnp.concatenate([h1s, jnp.zeros((pad, d), jnp.float32)],
                          axis=0).astype(jnp.bfloat16)
    p2 = _scatter_call(dst_col, _gather_call(src_col, h1p))[:n]
    out_row = _final_call(p2, nd_, W2, b2.reshape(1, d),
                          Wfc.reshape(n, 1), bfc.reshape(1, 1), bn)
    return out_row.reshape(d, 1)
